# baseline (speedup 1.0000x reference)
import functools

import jax
import jax.numpy as jnp
import numpy as np
from jax import lax
from jax.experimental import pallas as pl
from jax.experimental.pallas import tpu as pltpu
from jax.experimental.pallas import tpu_sc as plsc

_ROWS, _COLS = 16384, 1000
_NC, _NS = 2, 16
_NW = _NC * _NS
_RPW = _ROWS // _NW
_CH = 64
_CPW = _RPW // _CH
_BR = 512

_G = jax.random.gumbel(jax.random.key(1), (_ROWS, _COLS), dtype=jnp.float32)
_ZTILE = np.zeros((_CH * _COLS,), dtype=np.float32)


def _idx_body(x_ref, g_ref, o_ref):
    z = x_ref[...] + g_ref[...]
    m = jnp.max(z, axis=1, keepdims=True)
    col = lax.broadcasted_iota(jnp.int32, z.shape, 1)
    cand = jnp.where(z == m, col, _COLS)
    o_ref[...] = jnp.min(cand, axis=1, keepdims=True)


def _row_argmax(x):
    return pl.pallas_call(
        _idx_body,
        grid=(_ROWS // _BR,),
        in_specs=[
            pl.BlockSpec((_BR, _COLS), lambda i: (i, 0)),
            pl.BlockSpec((_BR, _COLS), lambda i: (i, 0)),
        ],
        out_specs=pl.BlockSpec((_BR, 1), lambda i: (i, 0)),
        out_shape=jax.ShapeDtypeStruct((_ROWS, 1), jnp.int32),
    )(x, _G)


@functools.partial(
    pl.kernel,
    out_type=jax.ShapeDtypeStruct((_ROWS * _COLS,), jnp.float32),
    mesh=plsc.VectorSubcoreMesh(core_axis_name="c", subcore_axis_name="s"),
    compiler_params=pltpu.CompilerParams(needs_layout_passes=False),
    scratch_types=[
        pltpu.VMEM((_CH * _COLS,), jnp.float32),
        pltpu.VMEM((_RPW,), jnp.int32),
        pltpu.SemaphoreType.DMA,
    ],
)
def _sc_onehot(idx_hbm, ztile_hbm, out_hbm, buf, idxv, sem):
    w = lax.axis_index("s") * _NC + lax.axis_index("c")
    row0 = w * _RPW
    pltpu.sync_copy(ztile_hbm, buf)
    pltpu.sync_copy(idx_hbm.at[pl.ds(row0, _RPW)], idxv)

    ones = jnp.ones((16,), jnp.float32)
    zeros = jnp.zeros((16,), jnp.float32)
    lane = lax.iota(jnp.int32, 16)
    loc_fix = jnp.ones((16,), jnp.int32)

    def put(k, vals):
        for t in range(_CH // 16):
            j = t * 16 + lane
            cols = idxv[pl.ds(k * _CH + t * 16, 16)]
            plsc.store_scatter(buf, [j * _COLS + cols], vals)

    prev = None
    for k in range(_CPW):
        if prev is not None:
            prev.wait()
            put(k - 1, zeros)
            if k - 1 == 0:
                @pl.when(w == 0)
                def _():
                    plsc.store_scatter(buf, [loc_fix], zeros)
        put(k, ones)
        if k == 0:
            @pl.when(w == 0)
            def _():
                plsc.store_scatter(buf, [loc_fix], ones)
        prev = pltpu.async_copy(
            buf, out_hbm.at[pl.ds((row0 + k * _CH) * _COLS, _CH * _COLS)], sem)
    prev.wait()


def kernel(x):
    idx = _row_argmax(x)
    return _sc_onehot(idx.reshape(_ROWS), _ZTILE).reshape(_ROWS, _COLS)


# speedup vs baseline: 1.2685x; 1.2685x over previous
import functools

import jax
import jax.numpy as jnp
import numpy as np
from jax import lax
from jax.experimental import pallas as pl
from jax.experimental.pallas import tpu as pltpu
from jax.experimental.pallas import tpu_sc as plsc

_ROWS, _COLS = 16384, 1000
_NC, _NS = 2, 16
_NW = _NC * _NS
_RPW = _ROWS // _NW
_CH = 64
_CPW = _RPW // _CH
_BR = 512

_G = jax.random.gumbel(jax.random.key(1), (_ROWS, _COLS), dtype=jnp.float32)
_ZTILE = np.zeros((_CH, _COLS), dtype=np.float32)


def _idx_body(x_ref, g_ref, o_ref):
    z = x_ref[...] + g_ref[...]
    m = jnp.max(z, axis=1, keepdims=True)
    col = lax.broadcasted_iota(jnp.int32, z.shape, 1)
    cand = jnp.where(z == m, col, _COLS)
    o_ref[...] = jnp.min(cand, axis=1, keepdims=True)


def _row_argmax(x):
    return pl.pallas_call(
        _idx_body,
        grid=(_ROWS // _BR,),
        in_specs=[
            pl.BlockSpec((_BR, _COLS), lambda i: (i, 0)),
            pl.BlockSpec((_BR, _COLS), lambda i: (i, 0)),
        ],
        out_specs=pl.BlockSpec((_BR, 1), lambda i: (i, 0)),
        out_shape=jax.ShapeDtypeStruct((_ROWS, 1), jnp.int32),
    )(x, _G)


@functools.partial(
    pl.kernel,
    out_type=jax.ShapeDtypeStruct((_ROWS, _COLS), jnp.float32),
    mesh=plsc.VectorSubcoreMesh(core_axis_name="c", subcore_axis_name="s"),
    compiler_params=pltpu.CompilerParams(needs_layout_passes=False),
    scratch_types=[
        pltpu.VMEM((_CH, _COLS), jnp.float32),
        pltpu.VMEM((_RPW,), jnp.int32),
        pltpu.SemaphoreType.DMA,
    ],
)
def _sc_onehot(idx_hbm, ztile_hbm, out_hbm, buf, idxv, sem):
    w = lax.axis_index("s") * _NC + lax.axis_index("c")
    row0 = w * _RPW
    pltpu.sync_copy(ztile_hbm, buf)
    pltpu.sync_copy(idx_hbm.at[pl.ds(row0, _RPW)], idxv)

    ones = jnp.ones((16,), jnp.float32)
    zeros = jnp.zeros((16,), jnp.float32)
    lane = lax.iota(jnp.int32, 16)
    row_fix = jnp.zeros((16,), jnp.int32)
    col_fix = jnp.ones((16,), jnp.int32)

    def put(k, vals):
        for t in range(_CH // 16):
            j = t * 16 + lane
            cols = idxv[pl.ds(k * _CH + t * 16, 16)]
            plsc.store_scatter(buf, [j, cols], vals)

    prev = None
    for k in range(_CPW):
        if prev is not None:
            prev.wait()
            put(k - 1, zeros)
            if k - 1 == 0:
                @pl.when(w == 0)
                def _():
                    plsc.store_scatter(buf, [row_fix, col_fix], zeros)
        put(k, ones)
        if k == 0:
            @pl.when(w == 0)
            def _():
                plsc.store_scatter(buf, [row_fix, col_fix], ones)
        prev = pltpu.async_copy(
            buf, out_hbm.at[pl.ds(row0 + k * _CH, _CH)], sem)
    prev.wait()


def kernel(x):
    idx = _row_argmax(x)
    return _sc_onehot(idx.reshape(_ROWS), _ZTILE)


# speedup vs baseline: 4.5227x; 3.5654x over previous
import jax
import jax.numpy as jnp
from jax import lax
from jax.experimental import pallas as pl

_ROWS, _COLS = 16384, 1000
_BN = 1024

_GT = jax.random.gumbel(jax.random.key(1), (_ROWS, _COLS), dtype=jnp.float32).T


def _onehot_t_body(xt_ref, gt_ref, o_ref):
    z = xt_ref[...] + gt_ref[...]
    m = jnp.max(z, axis=0, keepdims=True)
    row = lax.broadcasted_iota(jnp.int32, z.shape, 0)
    cand = jnp.where(z == m, row, _COLS)
    idx = jnp.min(cand, axis=0, keepdims=True)
    o_ref[...] = jnp.where(row == idx, 1.0, 0.0).astype(o_ref.dtype)

    @pl.when(pl.program_id(0) == 0)
    def _():
        o_ref[1:2, 0:1] = jnp.ones((1, 1), o_ref.dtype)


def kernel(x):
    out_t = pl.pallas_call(
        _onehot_t_body,
        grid=(_ROWS // _BN,),
        in_specs=[
            pl.BlockSpec((_COLS, _BN), lambda i: (0, i)),
            pl.BlockSpec((_COLS, _BN), lambda i: (0, i)),
        ],
        out_specs=pl.BlockSpec((_COLS, _BN), lambda i: (0, i)),
        out_shape=jax.ShapeDtypeStruct((_COLS, _ROWS), jnp.float32),
    )(x.T, _GT)
    return out_t.T
